# Initial kernel scaffold; baseline (speedup 1.0000x reference)
import jax
import jax.numpy as jnp
from jax.experimental import pallas as pl


def kernel(x, edge_index, eps, W1, b1, g1, be1, W2, b2, g2, be2):
    raise NotImplementedError("write your pallas kernel here")


# speedup vs baseline: 3.6483x; 3.6483x over previous
import functools

import jax
import jax.numpy as jnp
from jax import lax
from jax.experimental import pallas as pl
from jax.experimental.pallas import tpu as pltpu
from jax.experimental.pallas import tpu_sc as plsc

N_NODES = 10000
D_IN = 256
D_HID = 512
N_SC = 2
N_TILES = 16
CHUNK = 128
N_CHUNKS = 79
EDGES_PER_TILE = CHUNK * N_CHUNKS
E_PAD = EDGES_PER_TILE * N_TILES
ACC_ROWS = 10240
ROWS_PER_TILE = ACC_ROWS // N_TILES
HALF = D_IN // 2


def _sc_aggregate(xcat, src3, dst3):
    @functools.partial(
        pl.kernel,
        mesh=plsc.VectorSubcoreMesh(core_axis_name="c", subcore_axis_name="s"),
        out_type=jax.ShapeDtypeStruct((N_SC, ACC_ROWS, HALF), jnp.float32),
        scratch_types=[
            pltpu.VMEM((N_CHUNKS, CHUNK), jnp.int32),
            pltpu.VMEM((N_CHUNKS, CHUNK), jnp.int32),
            pltpu.VMEM((CHUNK, HALF), jnp.float32),
            pltpu.VMEM_SHARED((ACC_ROWS, HALF), jnp.float32),
            pltpu.SemaphoreType.DMA,
        ],
    )
    def k(xcat_hbm, src_hbm, dst_hbm, out_hbm, src_v, dst_v, rows_v, acc_sh, sem):
        c = lax.axis_index("c")
        s = lax.axis_index("s")
        pltpu.sync_copy(src_hbm.at[c, s], src_v)
        pltpu.sync_copy(dst_hbm.at[s], dst_v)

        def _zrow(i, _):
            def _zlane(l, _):
                rows_v[i, pl.ds(l * 16, 16)] = jnp.zeros((16,), jnp.float32)
                return 0
            return lax.fori_loop(0, HALF // 16, _zlane, 0)

        lax.fori_loop(0, CHUNK, _zrow, 0)
        for kk in range(ROWS_PER_TILE // CHUNK):
            pltpu.sync_copy(
                rows_v, acc_sh.at[pl.ds(s * ROWS_PER_TILE + kk * CHUNK, CHUNK)])
        plsc.subcore_barrier()

        def _step(j, _):
            pltpu.async_copy(xcat_hbm.at[src_v.at[j]], rows_v, sem).wait()
            pltpu.sync_copy(rows_v, acc_sh.at[dst_v.at[j]], add=True)
            return 0

        lax.fori_loop(0, N_CHUNKS, _step, 0)
        plsc.subcore_barrier()
        for kk in range(ROWS_PER_TILE // CHUNK):
            off = s * ROWS_PER_TILE + kk * CHUNK
            pltpu.sync_copy(acc_sh.at[pl.ds(off, CHUNK)],
                            out_hbm.at[c, pl.ds(off, CHUNK)])

    return k(xcat, src3, dst3)


_HIGH = jax.lax.Precision.HIGHEST


def _tc_layer1(epsv, x, aggr, W1, b1):
    blk = 1000

    def body(eps_ref, x_ref, aL_ref, aR_ref, w_ref, b_ref, y_ref, s_ref, q_ref):
        i = pl.program_id(0)
        e = eps_ref[0, 0]
        h = (1.0 + e) * x_ref[...] + jnp.concatenate(
            [aL_ref[0], aR_ref[0]], axis=1)
        y = jnp.dot(h, w_ref[...], preferred_element_type=jnp.float32,
                    precision=_HIGH) + b_ref[...]
        y_ref[...] = y

        @pl.when(i == 0)
        def _():
            s_ref[...] = jnp.zeros_like(s_ref)
            q_ref[...] = jnp.zeros_like(q_ref)

        s_ref[...] += jnp.sum(y, axis=0, keepdims=True)
        q_ref[...] += jnp.sum(y * y, axis=0, keepdims=True)

    return pl.pallas_call(
        body,
        grid=(N_NODES // blk,),
        in_specs=[
            pl.BlockSpec((1, 1), lambda i: (0, 0), memory_space=pltpu.SMEM),
            pl.BlockSpec((blk, D_IN), lambda i: (i, 0)),
            pl.BlockSpec((1, blk, HALF), lambda i: (0, i, 0)),
            pl.BlockSpec((1, blk, HALF), lambda i: (1, i, 0)),
            pl.BlockSpec((D_IN, D_HID), lambda i: (0, 0)),
            pl.BlockSpec((1, D_HID), lambda i: (0, 0)),
        ],
        out_specs=[
            pl.BlockSpec((blk, D_HID), lambda i: (i, 0)),
            pl.BlockSpec((1, D_HID), lambda i: (0, 0)),
            pl.BlockSpec((1, D_HID), lambda i: (0, 0)),
        ],
        out_shape=[
            jax.ShapeDtypeStruct((N_NODES, D_HID), jnp.float32),
            jax.ShapeDtypeStruct((1, D_HID), jnp.float32),
            jax.ShapeDtypeStruct((1, D_HID), jnp.float32),
        ],
    )(epsv, x, aggr, aggr, W1, b1)


def _tc_layer2(y1, A1, B1, W2, b2):
    blk = 1000

    def body(y_ref, a_ref, c_ref, w_ref, b_ref, y2_ref, s_ref, q_ref):
        i = pl.program_id(0)
        z = jnp.maximum(y_ref[...] * a_ref[...] + c_ref[...], 0.0)
        y2 = jnp.dot(z, w_ref[...], preferred_element_type=jnp.float32,
                     precision=_HIGH) + b_ref[...]
        y2_ref[...] = y2

        @pl.when(i == 0)
        def _():
            s_ref[...] = jnp.zeros_like(s_ref)
            q_ref[...] = jnp.zeros_like(q_ref)

        s_ref[...] += jnp.sum(y2, axis=0, keepdims=True)
        q_ref[...] += jnp.sum(y2 * y2, axis=0, keepdims=True)

    return pl.pallas_call(
        body,
        grid=(N_NODES // blk,),
        in_specs=[
            pl.BlockSpec((blk, D_HID), lambda i: (i, 0)),
            pl.BlockSpec((1, D_HID), lambda i: (0, 0)),
            pl.BlockSpec((1, D_HID), lambda i: (0, 0)),
            pl.BlockSpec((D_HID, D_HID), lambda i: (0, 0)),
            pl.BlockSpec((1, D_HID), lambda i: (0, 0)),
        ],
        out_specs=[
            pl.BlockSpec((blk, D_HID), lambda i: (i, 0)),
            pl.BlockSpec((1, D_HID), lambda i: (0, 0)),
            pl.BlockSpec((1, D_HID), lambda i: (0, 0)),
        ],
        out_shape=[
            jax.ShapeDtypeStruct((N_NODES, D_HID), jnp.float32),
            jax.ShapeDtypeStruct((1, D_HID), jnp.float32),
            jax.ShapeDtypeStruct((1, D_HID), jnp.float32),
        ],
    )(y1, A1, B1, W2, b2)


def _tc_layer3(y2, A2, B2):
    blk = 1000

    def body(y_ref, a_ref, c_ref, o_ref):
        o_ref[...] = jnp.maximum(y_ref[...] * a_ref[...] + c_ref[...], 0.0)

    return pl.pallas_call(
        body,
        grid=(N_NODES // blk,),
        in_specs=[
            pl.BlockSpec((blk, D_HID), lambda i: (i, 0)),
            pl.BlockSpec((1, D_HID), lambda i: (0, 0)),
            pl.BlockSpec((1, D_HID), lambda i: (0, 0)),
        ],
        out_specs=pl.BlockSpec((blk, D_HID), lambda i: (i, 0)),
        out_shape=jax.ShapeDtypeStruct((N_NODES, D_HID), jnp.float32),
    )(y2, A2, B2)


def kernel(x, edge_index, eps, W1, b1, g1, be1, W2, b2, g2, be2):
    E = edge_index.shape[1]
    src = edge_index[0]
    dst = edge_index[1]

    pad = E_PAD - E
    src_p = jnp.concatenate([src, jnp.zeros((pad,), jnp.int32)])
    dst_p = jnp.concatenate([dst, jnp.full((pad,), N_NODES, jnp.int32)])
    src3 = jnp.stack([src_p, src_p + N_NODES]).reshape(
        N_SC, N_TILES, N_CHUNKS, CHUNK)
    dst3 = dst_p.reshape(N_TILES, N_CHUNKS, CHUNK)
    xcat = jnp.concatenate([x[:, :HALF], x[:, HALF:]], axis=0)

    aggr = _sc_aggregate(xcat, src3, dst3)

    epsv = jnp.reshape(eps, (1, 1))
    y1, s1, q1 = _tc_layer1(epsv, x, aggr, W1, jnp.reshape(b1, (1, D_HID)))
    m1 = s1 / N_NODES
    v1 = q1 / N_NODES - m1 * m1
    A1 = g1 / jnp.sqrt(v1 + 1e-5)
    B1 = be1 - m1 * A1

    y2, s2, q2 = _tc_layer2(y1, A1, B1, W2, jnp.reshape(b2, (1, D_HID)))
    m2 = s2 / N_NODES
    v2 = q2 / N_NODES - m2 * m2
    A2 = g2 / jnp.sqrt(v2 + 1e-5)
    B2 = be2 - m2 * A2

    return _tc_layer3(y2, A2, B2)
